# baseline (speedup 1.0000x reference)
import jax
import jax.numpy as jnp
from jax.experimental import pallas as pl

_BM = 400


def _g1_kernel(x_ref, w1t_ref, b1_ref, o_ref):
    o_ref[...] = (
        jnp.dot(x_ref[...], w1t_ref[...], preferred_element_type=jnp.float32)
        + b1_ref[...]
    )


def _pass1_kernel(adj_ref, g1_ref, w2t_ref, b2_ref, o_ref):
    a = adj_ref[...].astype(jnp.bfloat16)
    g = g1_ref[...].astype(jnp.bfloat16)
    h = jnp.dot(a, g, preferred_element_type=jnp.float32)
    h = jnp.maximum(h, 0.0)
    o_ref[...] = (
        jnp.dot(h, w2t_ref[...], preferred_element_type=jnp.float32) + b2_ref[...]
    )


def _pass2_kernel(
    adj_ref, g2_ref, wmut_ref, bmu_ref, wlvt_ref, blv_ref,
    wp1t_ref, bp1_ref, wp2t_ref, bp2_ref, eps_ref,
    z_ref, xs_ref, mu_ref, lv_ref,
):
    a = adj_ref[...].astype(jnp.bfloat16)
    g = g2_ref[...].astype(jnp.bfloat16)
    h = jnp.dot(a, g, preferred_element_type=jnp.float32)
    mu = jnp.dot(h, wmut_ref[...], preferred_element_type=jnp.float32) + bmu_ref[...]
    lv = jnp.dot(h, wlvt_ref[...], preferred_element_type=jnp.float32) + blv_ref[...]
    std = jnp.exp(0.5 * lv)
    xs = mu + std * eps_ref[...]
    p = jnp.maximum(
        jnp.dot(xs, wp1t_ref[...], preferred_element_type=jnp.float32) + bp1_ref[...],
        0.0,
    )
    z = jnp.dot(p, wp2t_ref[...], preferred_element_type=jnp.float32) + bp2_ref[...]
    z_ref[...] = z
    xs_ref[...] = xs
    mu_ref[...] = mu
    lv_ref[...] = lv


def kernel(x, Adj, W1, b1, W2, b2, Wmu, bmu, Wlv, blv, Wp1, bp1, Wp2, bp2, eps):
    n, in_dim = x.shape
    hid = W1.shape[0]
    emb = W2.shape[0]
    zd = Wmu.shape[0]
    proj = Wp1.shape[0]

    w1t = W1.T
    w2t = W2.T
    wmut = Wmu.T
    wlvt = Wlv.T
    wp1t = Wp1.T
    wp2t = Wp2.T
    b1r = b1.reshape(1, hid)
    b2r = b2.reshape(1, emb)
    bmur = bmu.reshape(1, zd)
    blvr = blv.reshape(1, zd)
    bp1r = bp1.reshape(1, proj)
    bp2r = bp2.reshape(1, proj)

    g1 = pl.pallas_call(
        _g1_kernel,
        out_shape=jax.ShapeDtypeStruct((n, hid), jnp.float32),
    )(x, w1t, b1r)

    grid = (n // _BM,)
    full = lambda i: (0, 0)
    rowblk = lambda i: (i, 0)

    g2 = pl.pallas_call(
        _pass1_kernel,
        grid=grid,
        in_specs=[
            pl.BlockSpec((_BM, n), rowblk),
            pl.BlockSpec((n, hid), full),
            pl.BlockSpec((hid, emb), full),
            pl.BlockSpec((1, emb), full),
        ],
        out_specs=pl.BlockSpec((_BM, emb), rowblk),
        out_shape=jax.ShapeDtypeStruct((n, emb), jnp.float32),
    )(Adj, g1, w2t, b2r)

    z, xs, mu, lv = pl.pallas_call(
        _pass2_kernel,
        grid=grid,
        in_specs=[
            pl.BlockSpec((_BM, n), rowblk),
            pl.BlockSpec((n, emb), full),
            pl.BlockSpec((emb, zd), full),
            pl.BlockSpec((1, zd), full),
            pl.BlockSpec((emb, zd), full),
            pl.BlockSpec((1, zd), full),
            pl.BlockSpec((zd, proj), full),
            pl.BlockSpec((1, proj), full),
            pl.BlockSpec((proj, proj), full),
            pl.BlockSpec((1, proj), full),
            pl.BlockSpec((_BM, zd), rowblk),
        ],
        out_specs=[
            pl.BlockSpec((_BM, proj), rowblk),
            pl.BlockSpec((_BM, zd), rowblk),
            pl.BlockSpec((_BM, zd), rowblk),
            pl.BlockSpec((_BM, zd), rowblk),
        ],
        out_shape=[
            jax.ShapeDtypeStruct((n, proj), jnp.float32),
            jax.ShapeDtypeStruct((n, zd), jnp.float32),
            jax.ShapeDtypeStruct((n, zd), jnp.float32),
            jax.ShapeDtypeStruct((n, zd), jnp.float32),
        ],
    )(Adj, g2, wmut, bmur, wlvt, blvr, wp1t, bp1r, wp2t, bp2r, eps)

    return (z, xs, mu, lv)


# speedup vs baseline: 1.0489x; 1.0489x over previous
import jax
import jax.numpy as jnp
from jax.experimental import pallas as pl
from jax.experimental.pallas import tpu as pltpu

_BM = 400


def _fused_kernel(
    x_ref, adj_ref, w1t_ref, b1_ref, w2t_ref, b2_ref,
    wmut_ref, bmu_ref, wlvt_ref, blv_ref,
    wp1t_ref, bp1_ref, wp2t_ref, bp2_ref, eps_ref,
    z_ref, xs_ref, mu_ref, lv_ref,
    g1_scr, g2_scr,
):
    p = pl.program_id(0)
    i = pl.program_id(1)

    @pl.when(p == 0)
    def _phase0():
        @pl.when(i == 0)
        def _init():
            g1 = (
                jnp.dot(x_ref[...], w1t_ref[...], preferred_element_type=jnp.float32)
                + b1_ref[...]
            )
            g1_scr[...] = g1.astype(jnp.bfloat16)

        a = adj_ref[...].astype(jnp.bfloat16)
        h = jnp.dot(a, g1_scr[...], preferred_element_type=jnp.float32)
        h = jnp.maximum(h, 0.0)
        g2 = jnp.dot(h, w2t_ref[...], preferred_element_type=jnp.float32) + b2_ref[...]
        g2_scr[pl.ds(i * _BM, _BM), :] = g2.astype(jnp.bfloat16)

    @pl.when(p == 1)
    def _phase1():
        a = adj_ref[...].astype(jnp.bfloat16)
        h = jnp.dot(a, g2_scr[...], preferred_element_type=jnp.float32)
        mu = jnp.dot(h, wmut_ref[...], preferred_element_type=jnp.float32) + bmu_ref[...]
        lv = jnp.dot(h, wlvt_ref[...], preferred_element_type=jnp.float32) + blv_ref[...]
        std = jnp.exp(0.5 * lv)
        xs = mu + std * eps_ref[...]
        pr = jnp.maximum(
            jnp.dot(xs, wp1t_ref[...], preferred_element_type=jnp.float32) + bp1_ref[...],
            0.0,
        )
        z = jnp.dot(pr, wp2t_ref[...], preferred_element_type=jnp.float32) + bp2_ref[...]
        z_ref[...] = z
        xs_ref[...] = xs
        mu_ref[...] = mu
        lv_ref[...] = lv


def kernel(x, Adj, W1, b1, W2, b2, Wmu, bmu, Wlv, blv, Wp1, bp1, Wp2, bp2, eps):
    n, in_dim = x.shape
    hid = W1.shape[0]
    emb = W2.shape[0]
    zd = Wmu.shape[0]
    proj = Wp1.shape[0]

    w1t = W1.T
    w2t = W2.T
    wmut = Wmu.T
    wlvt = Wlv.T
    wp1t = Wp1.T
    wp2t = Wp2.T
    b1r = b1.reshape(1, hid)
    b2r = b2.reshape(1, emb)
    bmur = bmu.reshape(1, zd)
    blvr = blv.reshape(1, zd)
    bp1r = bp1.reshape(1, proj)
    bp2r = bp2.reshape(1, proj)

    full = lambda p, i: (0, 0)
    rowblk = lambda p, i: (i, 0)
    outblk = lambda p, i: (p * i, 0)

    z, xs, mu, lv = pl.pallas_call(
        _fused_kernel,
        grid=(2, n // _BM),
        in_specs=[
            pl.BlockSpec((n, in_dim), full),
            pl.BlockSpec((_BM, n), rowblk),
            pl.BlockSpec((in_dim, hid), full),
            pl.BlockSpec((1, hid), full),
            pl.BlockSpec((hid, emb), full),
            pl.BlockSpec((1, emb), full),
            pl.BlockSpec((emb, zd), full),
            pl.BlockSpec((1, zd), full),
            pl.BlockSpec((emb, zd), full),
            pl.BlockSpec((1, zd), full),
            pl.BlockSpec((zd, proj), full),
            pl.BlockSpec((1, proj), full),
            pl.BlockSpec((proj, proj), full),
            pl.BlockSpec((1, proj), full),
            pl.BlockSpec((_BM, zd), rowblk),
        ],
        out_specs=[
            pl.BlockSpec((_BM, proj), outblk),
            pl.BlockSpec((_BM, zd), outblk),
            pl.BlockSpec((_BM, zd), outblk),
            pl.BlockSpec((_BM, zd), outblk),
        ],
        out_shape=[
            jax.ShapeDtypeStruct((n, proj), jnp.float32),
            jax.ShapeDtypeStruct((n, zd), jnp.float32),
            jax.ShapeDtypeStruct((n, zd), jnp.float32),
            jax.ShapeDtypeStruct((n, zd), jnp.float32),
        ],
        scratch_shapes=[
            pltpu.VMEM((n, hid), jnp.bfloat16),
            pltpu.VMEM((n, emb), jnp.bfloat16),
        ],
    )(x, Adj, w1t, b1r, w2t, b2r, wmut, bmur, wlvt, blvr, wp1t, bp1r, wp2t, bp2r, eps)

    return (z, xs, mu, lv)


# speedup vs baseline: 1.0680x; 1.0182x over previous
import jax
import jax.numpy as jnp
from jax.experimental import pallas as pl
from jax.experimental.pallas import tpu as pltpu

_BM = 400

_NT = (((1,), (1,)), ((), ()))


def _fused_kernel(
    x_ref, adj_ref, w1_ref, b1_ref, w2_ref, b2_ref,
    wmu_ref, bmu_ref, wlv_ref, blv_ref,
    wp1_ref, bp1_ref, wp2_ref, bp2_ref, epst_ref,
    z_ref, xs_ref, mu_ref, lv_ref,
    g1t_scr, g2nm_scr, g2t_scr,
):
    p = pl.program_id(0)
    i = pl.program_id(1)

    @pl.when(p == 0)
    def _phase0():
        @pl.when(i == 0)
        def _init():
            g1t_scr[...] = (
                jax.lax.dot_general(
                    w1_ref[...], x_ref[...], _NT,
                    preferred_element_type=jnp.float32,
                )
                + b1_ref[...]
            )

        ht = jax.lax.dot_general(
            g1t_scr[...], adj_ref[...], _NT,
            preferred_element_type=jnp.float32,
        )
        ht = jnp.maximum(ht, 0.0)
        g2blk = (
            jnp.dot(w2_ref[...], ht, preferred_element_type=jnp.float32)
            + b2_ref[...]
        )
        g2nm_scr[pl.ds(i * _BM, _BM), :] = g2blk.T

    @pl.when(p == 1)
    def _phase1():
        @pl.when(i == 0)
        def _retile():
            g2t_scr[...] = g2nm_scr[...].T

        ht = jax.lax.dot_general(
            g2t_scr[...], adj_ref[...], _NT,
            preferred_element_type=jnp.float32,
        )
        mut = jnp.dot(wmu_ref[...], ht, preferred_element_type=jnp.float32) + bmu_ref[...]
        lvt = jnp.dot(wlv_ref[...], ht, preferred_element_type=jnp.float32) + blv_ref[...]
        stdt = jnp.exp(0.5 * lvt)
        xst = mut + stdt * epst_ref[0]
        pt = jnp.maximum(
            jnp.dot(wp1_ref[...], xst, preferred_element_type=jnp.float32) + bp1_ref[...],
            0.0,
        )
        zt = jnp.dot(wp2_ref[...], pt, preferred_element_type=jnp.float32) + bp2_ref[...]
        z_ref[...] = zt.T
        xs_ref[...] = xst.T
        mu_ref[...] = mut.T
        lv_ref[...] = lvt.T


def kernel(x, Adj, W1, b1, W2, b2, Wmu, bmu, Wlv, blv, Wp1, bp1, Wp2, bp2, eps):
    n, in_dim = x.shape
    hid = W1.shape[0]
    emb = W2.shape[0]
    zd = Wmu.shape[0]
    proj = Wp1.shape[0]
    nb = n // _BM

    b1c = b1.reshape(hid, 1)
    b2c = b2.reshape(emb, 1)
    bmuc = bmu.reshape(zd, 1)
    blvc = blv.reshape(zd, 1)
    bp1c = bp1.reshape(proj, 1)
    bp2c = bp2.reshape(proj, 1)
    epst3 = eps.reshape(nb, _BM, zd).transpose(0, 2, 1)

    full = lambda p, i: (0, 0)
    rowblk = lambda p, i: (i, 0)
    outblk = lambda p, i: (p * i, 0)

    z, xs, mu, lv = pl.pallas_call(
        _fused_kernel,
        grid=(2, nb),
        in_specs=[
            pl.BlockSpec((n, in_dim), full),
            pl.BlockSpec((_BM, n), rowblk),
            pl.BlockSpec((hid, in_dim), full),
            pl.BlockSpec((hid, 1), full),
            pl.BlockSpec((emb, hid), full),
            pl.BlockSpec((emb, 1), full),
            pl.BlockSpec((zd, emb), full),
            pl.BlockSpec((zd, 1), full),
            pl.BlockSpec((zd, emb), full),
            pl.BlockSpec((zd, 1), full),
            pl.BlockSpec((proj, zd), full),
            pl.BlockSpec((proj, 1), full),
            pl.BlockSpec((proj, proj), full),
            pl.BlockSpec((proj, 1), full),
            pl.BlockSpec((1, zd, _BM), lambda p, i: (i, 0, 0)),
        ],
        out_specs=[
            pl.BlockSpec((_BM, proj), outblk),
            pl.BlockSpec((_BM, zd), outblk),
            pl.BlockSpec((_BM, zd), outblk),
            pl.BlockSpec((_BM, zd), outblk),
        ],
        out_shape=[
            jax.ShapeDtypeStruct((n, proj), jnp.float32),
            jax.ShapeDtypeStruct((n, zd), jnp.float32),
            jax.ShapeDtypeStruct((n, zd), jnp.float32),
            jax.ShapeDtypeStruct((n, zd), jnp.float32),
        ],
        scratch_shapes=[
            pltpu.VMEM((hid, n), jnp.float32),
            pltpu.VMEM((n, emb), jnp.float32),
            pltpu.VMEM((emb, n), jnp.float32),
        ],
    )(x, Adj, W1, b1c, W2, b2c, Wmu, bmuc, Wlv, blvc, Wp1, bp1c, Wp2, bp2c, epst3)

    return (z, xs, mu, lv)
